# baseline (speedup 1.0000x reference)
import jax
import jax.numpy as jnp
from jax import lax
from jax.experimental import pallas as pl
from jax.experimental.pallas import tpu as pltpu
from jax.experimental.pallas import tpu_sc as plsc

N_USERS = 20000
N_ITEMS = 30000
N_NODES = N_USERS + N_ITEMS
N_EDGES = 800000
D = 64

HALF = N_NODES // 2
ACC_ROWS = 25088
E_PAD = 819200
EROWS = E_PAD // 128
IN_BLOCKS_PER_W = EROWS // 32
BUCKET_BLOCKS = 200
BUCKET_STRIDE = BUCKET_BLOCKS * 384
PPACK_SIZE = 64 * BUCKET_STRIDE
STRIPE = 1568
LAST_STRIPE = HALF - 15 * STRIPE


def _part_body(packed_hbm, ppack_hbm, cnt_hbm,
               pbuf, stg0s, stg0d, stg0v, stg1s, stg1d, stg1v,
               cstg, ptrs):
    c = lax.axis_index("c")
    s = lax.axis_index("s")
    w = c * 16 + s
    stg = ((stg0s, stg0d, stg0v), (stg1s, stg1d, stg1v))

    ptrs[0] = 0
    ptrs[1] = 0
    ptrs[2] = 0
    ptrs[3] = 0
    iota16 = lax.iota(jnp.int32, 16)

    def in_body(t, carry):
        b3 = pl.multiple_of((w * IN_BLOCKS_PER_W + t) * 3, 3)
        pltpu.sync_copy(packed_hbm.at[pl.ds(b3, 3)], pbuf)
        for i in range(8):
            s_v = pbuf[0, pl.ds(i * 16, 16)]
            d_v = pbuf[1, pl.ds(i * 16, 16)]
            v_v = pbuf[2, pl.ds(i * 16, 16)]
            m0 = d_v < HALF
            for h in range(2):
                mh = m0 if h == 0 else jnp.logical_not(m0)
                dloc = d_v - h * HALF
                mi = jnp.where(mh, 1, 0).astype(jnp.int32)
                pc = plsc.cumsum(mi)
                p = ptrs[h]
                pos = p + pc - mi
                plsc.store_scatter(stg[h][0], [pos], s_v, mask=mh)
                plsc.store_scatter(stg[h][1], [pos], dloc, mask=mh)
                plsc.store_scatter(stg[h][2], [pos], v_v, mask=mh)
                ptrs[h] = p + pc[15]
        for h in range(2):
            @pl.when(ptrs[h] >= 128)
            def _(h=h):
                blk = ptrs[2 + h]
                base = (h * 32 + w) * BUCKET_STRIDE + blk * 384
                base = pl.multiple_of(base, 128)
                for a in range(3):
                    pltpu.sync_copy(stg[h][a].at[pl.ds(0, 128)],
                                    ppack_hbm.at[pl.ds(base + a * 128, 128)])
                    for i in range(8):
                        stg[h][a][pl.ds(i * 16, 16)] = (
                            stg[h][a][pl.ds(128 + i * 16, 16)])
                ptrs[h] = ptrs[h] - 128
                ptrs[2 + h] = blk + 1
        return carry

    lax.fori_loop(0, IN_BLOCKS_PER_W, in_body, 0)

    for h in range(2):
        p = ptrs[h]
        for i in range(8):
            keep = (i * 16 + iota16) < p
            for a in range(3):
                x = stg[h][a][pl.ds(i * 16, 16)]
                stg[h][a][pl.ds(i * 16, 16)] = jnp.where(keep, x, 0)

        @pl.when(p > 0)
        def _(h=h):
            blk = ptrs[2 + h]
            base = (h * 32 + w) * BUCKET_STRIDE + blk * 384
            base = pl.multiple_of(base, 128)
            for a in range(3):
                pltpu.sync_copy(stg[h][a].at[pl.ds(0, 128)],
                                ppack_hbm.at[pl.ds(base + a * 128, 128)])

        nblk = ptrs[2 + h] + jnp.where(p > 0, 1, 0)
        cstg[pl.ds(0, 16)] = jnp.full((16,), 0, jnp.int32) + nblk
        pltpu.sync_copy(cstg, cnt_hbm.at[pl.ds((h * 32 + w) * 16, 16)])


@jax.jit
def _partition(packed):
    mesh = plsc.VectorSubcoreMesh(core_axis_name="c", subcore_axis_name="s")
    fn = pl.kernel(
        _part_body,
        mesh=mesh,
        out_type=(jax.ShapeDtypeStruct((PPACK_SIZE,), jnp.int32),
                  jax.ShapeDtypeStruct((1024,), jnp.int32)),
        scratch_types=[
            pltpu.VMEM((3, 128), jnp.int32),
            pltpu.VMEM((256,), jnp.int32),
            pltpu.VMEM((256,), jnp.int32),
            pltpu.VMEM((256,), jnp.int32),
            pltpu.VMEM((256,), jnp.int32),
            pltpu.VMEM((256,), jnp.int32),
            pltpu.VMEM((256,), jnp.int32),
            pltpu.VMEM((16,), jnp.int32),
            pltpu.SMEM((4,), jnp.int32),
        ],
        compiler_params=pltpu.CompilerParams(use_tc_tiling_on_sc=False,
                                             needs_layout_passes=False),
    )
    return fn(packed)


def _layer_body(ego_hbm, ppack_hbm, cnt_hbm, out_hbm,
                pbufA, pbufB, pbufC, rowsA, rowsB, rowsC,
                dstlA, dstlB, dstlC, cb1, cb2,
                acc, semA, semB, semC, ssemA, ssemB, ssemC):
    c = lax.axis_index("c")
    s = lax.axis_index("s")
    coff = c * HALF

    pbuf = (pbufA, pbufB, pbufC)
    rows = (rowsA, rowsB, rowsC)
    dstl = (dstlA, dstlB, dstlC)
    sem = (semA, semB, semC)
    ssem = (ssemA, ssemB, ssemC)

    z = jnp.zeros((16,), jnp.float32)

    def zero_body(g, carry):
        for r in range(16):
            for q in range(D // 16):
                rowsA[g * 16 + r, pl.ds(q * 16, 16)] = z
        return carry

    lax.fori_loop(0, 8, zero_body, 0)
    for m in range(12):
        pltpu.sync_copy(rowsA, acc.at[pl.ds(s * STRIPE + m * 128, 128)])
    pltpu.sync_copy(rowsA.at[pl.ds(0, 32)],
                    acc.at[pl.ds(s * STRIPE + 1536, 32)])

    pltpu.sync_copy(cnt_hbm.at[pl.ds((c * 32 + s) * 16, 16)], cb1)
    pltpu.sync_copy(cnt_hbm.at[pl.ds((c * 32 + s + 16) * 16, 16)], cb2)
    n1 = cb1[pl.ds(0, 16)][0]
    n2 = cb2[pl.ds(0, 16)][0]
    nt = n1 + n2
    base1 = (c * 32 + s) * BUCKET_STRIDE
    base2 = (c * 32 + s + 16) * BUCKET_STRIDE
    plsc.subcore_barrier()

    def blockbase(j):
        return pl.multiple_of(
            jnp.where(j < n1, base1 + j * 384, base2 + (j - n1) * 384), 128)

    def gather_handle(B):
        return pltpu.make_async_copy(ego_hbm.at[pbuf[B].at[pl.ds(0, 128)]],
                                     rows[B], sem[B])

    def scatter_handle(B):
        return pltpu.make_async_copy(rows[B], acc.at[dstl[B]], ssem[B])

    def stage_load(jn, B):
        jn = jnp.int32(jn)

        @pl.when((jn >= 3) & (jn < nt))
        def _():
            scatter_handle(B).wait()

        @pl.when(jn < nt)
        def _():
            base = blockbase(jn)
            pltpu.sync_copy(ppack_hbm.at[pl.ds(base, 384)], pbuf[B])
            for i in range(8):
                dstl[B][pl.ds(i * 16, 16)] = pbuf[B][pl.ds(128 + i * 16, 16)]
            gather_handle(B).start()

    def stage_proc(jn, B):
        @pl.when(jnp.int32(jn) < nt)
        def _():
            gather_handle(B).wait()
            for i in range(8):
                vv = plsc.bitcast(pbuf[B][pl.ds(256 + i * 16, 16)],
                                  jnp.float32)
                for l in range(16):
                    r = i * 16 + l
                    v = vv[l]
                    for u in range(D // 16):
                        rows[B][r, pl.ds(u * 16, 16)] = (
                            rows[B][r, pl.ds(u * 16, 16)] * v)
            scatter_handle(B).start(add=True)

    stage_load(0, 0)
    stage_load(1, 1)

    def ring_body(t, carry):
        stage_load(3 * t + 2, 2)
        stage_proc(3 * t, 0)
        stage_load(3 * t + 3, 0)
        stage_proc(3 * t + 1, 1)
        stage_load(3 * t + 4, 1)
        stage_proc(3 * t + 2, 2)
        return carry

    lax.fori_loop(0, (nt + 2) // 3, ring_body, 0)

    @pl.when(nt >= 1)
    def _():
        scatter_handle(0).wait()

    @pl.when(nt >= 2)
    def _():
        scatter_handle(1).wait()

    @pl.when(nt >= 3)
    def _():
        scatter_handle(2).wait()

    plsc.subcore_barrier()

    st = s * STRIPE

    @pl.when(s < 15)
    def _():
        pltpu.sync_copy(acc.at[pl.ds(st, STRIPE)],
                        out_hbm.at[pl.ds(coff + st, STRIPE)])

    @pl.when(s == 15)
    def _():
        pltpu.sync_copy(acc.at[pl.ds(15 * STRIPE, LAST_STRIPE)],
                        out_hbm.at[pl.ds(coff + 15 * STRIPE, LAST_STRIPE)])


@jax.jit
def _propagate_layer(ego, ppack, counts):
    mesh = plsc.VectorSubcoreMesh(core_axis_name="c", subcore_axis_name="s")
    fn = pl.kernel(
        _layer_body,
        mesh=mesh,
        out_type=jax.ShapeDtypeStruct((N_NODES, D), jnp.float32),
        scratch_types=[
            pltpu.VMEM((384,), jnp.int32),
            pltpu.VMEM((384,), jnp.int32),
            pltpu.VMEM((384,), jnp.int32),
            pltpu.VMEM((128, D), jnp.float32),
            pltpu.VMEM((128, D), jnp.float32),
            pltpu.VMEM((128, D), jnp.float32),
            pltpu.VMEM((128,), jnp.int32),
            pltpu.VMEM((128,), jnp.int32),
            pltpu.VMEM((128,), jnp.int32),
            pltpu.VMEM((16,), jnp.int32),
            pltpu.VMEM((16,), jnp.int32),
            pltpu.VMEM_SHARED((ACC_ROWS, D), jnp.float32),
            pltpu.SemaphoreType.DMA,
            pltpu.SemaphoreType.DMA,
            pltpu.SemaphoreType.DMA,
            pltpu.SemaphoreType.DMA,
            pltpu.SemaphoreType.DMA,
            pltpu.SemaphoreType.DMA,
        ],
        compiler_params=pltpu.CompilerParams(use_tc_tiling_on_sc=False,
                                             needs_layout_passes=False),
    )
    return fn(ego, ppack, counts)


def _mean_body(a, b, c, d, o):
    o[...] = (a[...] + b[...] + c[...] + d[...]) * 0.25


@jax.jit
def _mean4(a, b, c, d):
    spec = pl.BlockSpec((400, D), lambda i: (i, 0))
    return pl.pallas_call(
        _mean_body,
        grid=(N_NODES // 400,),
        in_specs=[spec] * 4,
        out_specs=spec,
        out_shape=jax.ShapeDtypeStruct((N_NODES, D), jnp.float32),
    )(a, b, c, d)


@jax.jit
def _pack_edges(adj_indices, adj_values):
    pad = E_PAD - N_EDGES
    src2d = jnp.concatenate(
        [adj_indices[0], jnp.zeros((pad,), jnp.int32)]).reshape(EROWS, 128)
    dst2d = jnp.concatenate(
        [adj_indices[1], jnp.zeros((pad,), jnp.int32)]).reshape(EROWS, 128)
    val2d = jnp.concatenate(
        [adj_values, jnp.zeros((pad,), jnp.float32)]
    ).reshape(EROWS, 128).view(jnp.int32)
    return jnp.stack([src2d, dst2d, val2d], axis=1).reshape(EROWS * 3, 128)


def kernel(user_emb, item_emb, adj_indices, adj_values):
    ego0 = jnp.concatenate([user_emb, item_emb], axis=0)
    packed = _pack_edges(adj_indices, adj_values)
    ppack, counts = _partition(packed)
    e1 = _propagate_layer(ego0, ppack, counts)
    e2 = _propagate_layer(e1, ppack, counts)
    e3 = _propagate_layer(e2, ppack, counts)
    final = _mean4(ego0, e1, e2, e3)
    return final[:N_USERS], final[N_USERS:]


# speedup vs baseline: 1.3521x; 1.3521x over previous
import jax
import jax.numpy as jnp
from jax import lax
from jax.experimental import pallas as pl
from jax.experimental.pallas import tpu as pltpu
from jax.experimental.pallas import tpu_sc as plsc

N_USERS = 20000
N_ITEMS = 30000
N_NODES = N_USERS + N_ITEMS
N_EDGES = 800000
D = 64

HALF = N_NODES // 2
ACC_ROWS = 25088
E_PAD = 819200
EROWS = E_PAD // 128
IN_BLOCKS_PER_W = EROWS // 32
BUCKET_BLOCKS = 200
BUCKET_STRIDE = BUCKET_BLOCKS * 384
PPACK_SIZE = 64 * BUCKET_STRIDE
STRIPE = 1568
LAST_STRIPE = HALF - 15 * STRIPE


def _part_body(packed_hbm, ppack_hbm, cnt_hbm,
               pbuf, stg0s, stg0d, stg0v, stg1s, stg1d, stg1v,
               cstg, ptrs):
    c = lax.axis_index("c")
    s = lax.axis_index("s")
    w = c * 16 + s
    stg = ((stg0s, stg0d, stg0v), (stg1s, stg1d, stg1v))

    ptrs[0] = 0
    ptrs[1] = 0
    ptrs[2] = 0
    ptrs[3] = 0
    iota16 = lax.iota(jnp.int32, 16)

    def in_body(t, carry):
        b3 = pl.multiple_of((w * IN_BLOCKS_PER_W + t) * 3, 3)
        pltpu.sync_copy(packed_hbm.at[pl.ds(b3, 3)], pbuf)
        for i in range(8):
            s_v = pbuf[0, pl.ds(i * 16, 16)]
            d_v = pbuf[1, pl.ds(i * 16, 16)]
            v_v = pbuf[2, pl.ds(i * 16, 16)]
            m0 = d_v < HALF
            for h in range(2):
                mh = m0 if h == 0 else jnp.logical_not(m0)
                dloc = d_v - h * HALF
                mi = jnp.where(mh, 1, 0).astype(jnp.int32)
                pc = plsc.cumsum(mi)
                p = ptrs[h]
                pos = p + pc - mi
                plsc.store_scatter(stg[h][0], [pos], s_v, mask=mh)
                plsc.store_scatter(stg[h][1], [pos], dloc, mask=mh)
                plsc.store_scatter(stg[h][2], [pos], v_v, mask=mh)
                ptrs[h] = p + pc[15]
        for h in range(2):
            @pl.when(ptrs[h] >= 128)
            def _(h=h):
                blk = ptrs[2 + h]
                base = (h * 32 + w) * BUCKET_STRIDE + blk * 384
                base = pl.multiple_of(base, 128)
                for a in range(3):
                    pltpu.sync_copy(stg[h][a].at[pl.ds(0, 128)],
                                    ppack_hbm.at[pl.ds(base + a * 128, 128)])
                    for i in range(8):
                        stg[h][a][pl.ds(i * 16, 16)] = (
                            stg[h][a][pl.ds(128 + i * 16, 16)])
                ptrs[h] = ptrs[h] - 128
                ptrs[2 + h] = blk + 1
        return carry

    lax.fori_loop(0, IN_BLOCKS_PER_W, in_body, 0)

    for h in range(2):
        p = ptrs[h]
        for i in range(8):
            keep = (i * 16 + iota16) < p
            for a in range(3):
                x = stg[h][a][pl.ds(i * 16, 16)]
                stg[h][a][pl.ds(i * 16, 16)] = jnp.where(keep, x, 0)

        @pl.when(p > 0)
        def _(h=h):
            blk = ptrs[2 + h]
            base = (h * 32 + w) * BUCKET_STRIDE + blk * 384
            base = pl.multiple_of(base, 128)
            for a in range(3):
                pltpu.sync_copy(stg[h][a].at[pl.ds(0, 128)],
                                ppack_hbm.at[pl.ds(base + a * 128, 128)])

        nblk = ptrs[2 + h] + jnp.where(p > 0, 1, 0)
        cstg[pl.ds(0, 16)] = jnp.full((16,), 0, jnp.int32) + nblk
        pltpu.sync_copy(cstg, cnt_hbm.at[pl.ds((h * 32 + w) * 16, 16)])


@jax.jit
def _partition(packed):
    mesh = plsc.VectorSubcoreMesh(core_axis_name="c", subcore_axis_name="s")
    fn = pl.kernel(
        _part_body,
        mesh=mesh,
        out_type=(jax.ShapeDtypeStruct((PPACK_SIZE,), jnp.int32),
                  jax.ShapeDtypeStruct((1024,), jnp.int32)),
        scratch_types=[
            pltpu.VMEM((3, 128), jnp.int32),
            pltpu.VMEM((256,), jnp.int32),
            pltpu.VMEM((256,), jnp.int32),
            pltpu.VMEM((256,), jnp.int32),
            pltpu.VMEM((256,), jnp.int32),
            pltpu.VMEM((256,), jnp.int32),
            pltpu.VMEM((256,), jnp.int32),
            pltpu.VMEM((16,), jnp.int32),
            pltpu.SMEM((4,), jnp.int32),
        ],
        compiler_params=pltpu.CompilerParams(use_tc_tiling_on_sc=False,
                                             needs_layout_passes=False),
    )
    return fn(packed)


def _layer_body(egob_hbm, ppack_hbm, cnt_hbm, out_hbm,
                pbufA, pbufB, pbufC, rbA, rbB, rbC, rowsA, rowsB,
                dstlA, dstlB, dstlC, cb1, cb2,
                acc, semA, semB, semC):
    c = lax.axis_index("c")
    s = lax.axis_index("s")
    coff = c * HALF

    pbuf = (pbufA, pbufB, pbufC)
    rb = (rbA, rbB, rbC)
    rows = (rowsA, rowsB)
    dstl = (dstlA, dstlB, dstlC)
    sem = (semA, semB, semC)

    z = jnp.zeros((16,), jnp.float32)

    def zero_body(g, carry):
        for r in range(16):
            for q in range(D // 16):
                rowsA[g * 16 + r, pl.ds(q * 16, 16)] = z
        return carry

    lax.fori_loop(0, 8, zero_body, 0)
    for m in range(12):
        pltpu.sync_copy(rowsA, acc.at[pl.ds(s * STRIPE + m * 128, 128)])
    pltpu.sync_copy(rowsA.at[pl.ds(0, 32)],
                    acc.at[pl.ds(s * STRIPE + 1536, 32)])

    pltpu.sync_copy(cnt_hbm.at[pl.ds((c * 32 + s) * 16, 16)], cb1)
    pltpu.sync_copy(cnt_hbm.at[pl.ds((c * 32 + s + 16) * 16, 16)], cb2)
    n1 = cb1[pl.ds(0, 16)][0]
    n2 = cb2[pl.ds(0, 16)][0]
    nt = n1 + n2
    base1 = (c * 32 + s) * BUCKET_STRIDE
    base2 = (c * 32 + s + 16) * BUCKET_STRIDE
    plsc.subcore_barrier()

    def blockbase(j):
        return pl.multiple_of(
            jnp.where(j < n1, base1 + j * 384, base2 + (j - n1) * 384), 128)

    def gather_handle(B):
        return pltpu.make_async_copy(egob_hbm.at[pbuf[B].at[pl.ds(0, 128)]],
                                     rb[B], sem[B])

    def stage_load(jn, B):
        @pl.when(jnp.int32(jn) < nt)
        def _():
            base = blockbase(jn)
            pltpu.sync_copy(ppack_hbm.at[pl.ds(base, 384)], pbuf[B])
            for i in range(8):
                dstl[B][pl.ds(i * 16, 16)] = pbuf[B][pl.ds(128 + i * 16, 16)]
            gather_handle(B).start()

    MASKHI = jnp.int32(-65536)

    def stage_proc(jn, B, F):
        @pl.when(jnp.int32(jn) < nt)
        def _():
            gather_handle(B).wait()
            for i in range(8):
                vv = plsc.bitcast(pbuf[B][pl.ds(256 + i * 16, 16)],
                                  jnp.float32)
                for l in range(16):
                    r = i * 16 + l
                    v = vv[l]
                    p0 = rb[B][r, pl.ds(0, 16)]
                    p1 = rb[B][r, pl.ds(16, 16)]
                    rows[F][r, pl.ds(0, 16)] = (
                        plsc.bitcast(lax.shift_left(p0, 16), jnp.float32) * v)
                    rows[F][r, pl.ds(16, 16)] = (
                        plsc.bitcast(lax.shift_left(p1, 16), jnp.float32) * v)
                    rows[F][r, pl.ds(32, 16)] = (
                        plsc.bitcast(p0 & MASKHI, jnp.float32) * v)
                    rows[F][r, pl.ds(48, 16)] = (
                        plsc.bitcast(p1 & MASKHI, jnp.float32) * v)
            pltpu.sync_copy(rows[F], acc.at[dstl[B]], add=True)

    stage_load(0, 0)
    stage_load(1, 1)

    def ring_body(t, carry):
        stage_load(3 * t + 2, 2)
        stage_proc(3 * t, 0, 0)
        stage_load(3 * t + 3, 0)
        stage_proc(3 * t + 1, 1, 1)
        stage_load(3 * t + 4, 1)
        stage_proc(3 * t + 2, 2, 0)
        return carry

    lax.fori_loop(0, (nt + 2) // 3, ring_body, 0)
    plsc.subcore_barrier()

    st = s * STRIPE

    @pl.when(s < 15)
    def _():
        pltpu.sync_copy(acc.at[pl.ds(st, STRIPE)],
                        out_hbm.at[pl.ds(coff + st, STRIPE)])

    @pl.when(s == 15)
    def _():
        pltpu.sync_copy(acc.at[pl.ds(15 * STRIPE, LAST_STRIPE)],
                        out_hbm.at[pl.ds(coff + 15 * STRIPE, LAST_STRIPE)])


@jax.jit
def _propagate_layer(egob, ppack, counts):
    mesh = plsc.VectorSubcoreMesh(core_axis_name="c", subcore_axis_name="s")
    fn = pl.kernel(
        _layer_body,
        mesh=mesh,
        out_type=jax.ShapeDtypeStruct((N_NODES, D), jnp.float32),
        scratch_types=[
            pltpu.VMEM((384,), jnp.int32),
            pltpu.VMEM((384,), jnp.int32),
            pltpu.VMEM((384,), jnp.int32),
            pltpu.VMEM((128, 32), jnp.int32),
            pltpu.VMEM((128, 32), jnp.int32),
            pltpu.VMEM((128, 32), jnp.int32),
            pltpu.VMEM((128, D), jnp.float32),
            pltpu.VMEM((128, D), jnp.float32),
            pltpu.VMEM((128,), jnp.int32),
            pltpu.VMEM((128,), jnp.int32),
            pltpu.VMEM((128,), jnp.int32),
            pltpu.VMEM((16,), jnp.int32),
            pltpu.VMEM((16,), jnp.int32),
            pltpu.VMEM_SHARED((ACC_ROWS, D), jnp.float32),
            pltpu.SemaphoreType.DMA,
            pltpu.SemaphoreType.DMA,
            pltpu.SemaphoreType.DMA,
        ],
        compiler_params=pltpu.CompilerParams(use_tc_tiling_on_sc=False,
                                             needs_layout_passes=False),
    )
    return fn(egob, ppack, counts)


def _mean_body(a, b, c, d, o):
    o[...] = (a[...] + b[...] + c[...] + d[...]) * 0.25


@jax.jit
def _mean4(a, b, c, d):
    spec = pl.BlockSpec((400, D), lambda i: (i, 0))
    return pl.pallas_call(
        _mean_body,
        grid=(N_NODES // 400,),
        in_specs=[spec] * 4,
        out_specs=spec,
        out_shape=jax.ShapeDtypeStruct((N_NODES, D), jnp.float32),
    )(a, b, c, d)


def _pack_bf16_body(x, o):
    xb = x[...].astype(jnp.bfloat16)
    b16 = jax.lax.bitcast_convert_type(xb, jnp.uint16).astype(jnp.uint32)
    p = b16[:, :32] | (b16[:, 32:] << 16)
    o[...] = p.astype(jnp.int32)


@jax.jit
def _pack_bf16(x):
    return pl.pallas_call(
        _pack_bf16_body,
        grid=(N_NODES // 400,),
        in_specs=[pl.BlockSpec((400, D), lambda i: (i, 0))],
        out_specs=pl.BlockSpec((400, D // 2), lambda i: (i, 0)),
        out_shape=jax.ShapeDtypeStruct((N_NODES, D // 2), jnp.int32),
    )(x)


@jax.jit
def _pack_edges(adj_indices, adj_values):
    pad = E_PAD - N_EDGES
    src2d = jnp.concatenate(
        [adj_indices[0], jnp.zeros((pad,), jnp.int32)]).reshape(EROWS, 128)
    dst2d = jnp.concatenate(
        [adj_indices[1], jnp.zeros((pad,), jnp.int32)]).reshape(EROWS, 128)
    val2d = jnp.concatenate(
        [adj_values, jnp.zeros((pad,), jnp.float32)]
    ).reshape(EROWS, 128).view(jnp.int32)
    return jnp.stack([src2d, dst2d, val2d], axis=1).reshape(EROWS * 3, 128)


def kernel(user_emb, item_emb, adj_indices, adj_values):
    ego0 = jnp.concatenate([user_emb, item_emb], axis=0)
    packed = _pack_edges(adj_indices, adj_values)
    ppack, counts = _partition(packed)
    e1 = _propagate_layer(_pack_bf16(ego0), ppack, counts)
    e2 = _propagate_layer(_pack_bf16(e1), ppack, counts)
    e3 = _propagate_layer(_pack_bf16(e2), ppack, counts)
    final = _mean4(ego0, e1, e2, e3)
    return final[:N_USERS], final[N_USERS:]


# speedup vs baseline: 1.3524x; 1.0002x over previous
import jax
import jax.numpy as jnp
from jax import lax
from jax.experimental import pallas as pl
from jax.experimental.pallas import tpu as pltpu
from jax.experimental.pallas import tpu_sc as plsc

N_USERS = 20000
N_ITEMS = 30000
N_NODES = N_USERS + N_ITEMS
N_EDGES = 800000
D = 64

HALF = N_NODES // 2
ACC_ROWS = 25088
E_PAD = 819200
EROWS = E_PAD // 128
IN_BLOCKS_PER_W = EROWS // 32
BUCKET_BLOCKS = 200
BUCKET_STRIDE = BUCKET_BLOCKS * 384
PPACK_SIZE = 64 * BUCKET_STRIDE
STRIPE = 1568
LAST_STRIPE = HALF - 15 * STRIPE


def _part_body(packed_hbm, ppack_hbm, cnt_hbm,
               pbuf, stg0s, stg0d, stg0v, stg1s, stg1d, stg1v,
               cstg, ptrs):
    c = lax.axis_index("c")
    s = lax.axis_index("s")
    w = c * 16 + s
    stg = ((stg0s, stg0d, stg0v), (stg1s, stg1d, stg1v))

    ptrs[0] = 0
    ptrs[1] = 0
    ptrs[2] = 0
    ptrs[3] = 0
    iota16 = lax.iota(jnp.int32, 16)

    def in_body(t, carry):
        b3 = pl.multiple_of((w * IN_BLOCKS_PER_W + t) * 3, 3)
        pltpu.sync_copy(packed_hbm.at[pl.ds(b3, 3)], pbuf)
        for i in range(8):
            s_v = pbuf[0, pl.ds(i * 16, 16)]
            d_v = pbuf[1, pl.ds(i * 16, 16)]
            v_v = pbuf[2, pl.ds(i * 16, 16)]
            m0 = d_v < HALF
            for h in range(2):
                mh = m0 if h == 0 else jnp.logical_not(m0)
                dloc = d_v - h * HALF
                mi = jnp.where(mh, 1, 0).astype(jnp.int32)
                pc = plsc.cumsum(mi)
                p = ptrs[h]
                pos = p + pc - mi
                plsc.store_scatter(stg[h][0], [pos], s_v, mask=mh)
                plsc.store_scatter(stg[h][1], [pos], dloc, mask=mh)
                plsc.store_scatter(stg[h][2], [pos], v_v, mask=mh)
                ptrs[h] = p + pc[15]
        for h in range(2):
            @pl.when(ptrs[h] >= 128)
            def _(h=h):
                blk = ptrs[2 + h]
                base = (h * 32 + w) * BUCKET_STRIDE + blk * 384
                base = pl.multiple_of(base, 128)
                for a in range(3):
                    pltpu.sync_copy(stg[h][a].at[pl.ds(0, 128)],
                                    ppack_hbm.at[pl.ds(base + a * 128, 128)])
                    for i in range(8):
                        stg[h][a][pl.ds(i * 16, 16)] = (
                            stg[h][a][pl.ds(128 + i * 16, 16)])
                ptrs[h] = ptrs[h] - 128
                ptrs[2 + h] = blk + 1
        return carry

    lax.fori_loop(0, IN_BLOCKS_PER_W, in_body, 0)

    for h in range(2):
        p = ptrs[h]
        for i in range(8):
            keep = (i * 16 + iota16) < p
            for a in range(3):
                x = stg[h][a][pl.ds(i * 16, 16)]
                stg[h][a][pl.ds(i * 16, 16)] = jnp.where(keep, x, 0)

        @pl.when(p > 0)
        def _(h=h):
            blk = ptrs[2 + h]
            base = (h * 32 + w) * BUCKET_STRIDE + blk * 384
            base = pl.multiple_of(base, 128)
            for a in range(3):
                pltpu.sync_copy(stg[h][a].at[pl.ds(0, 128)],
                                ppack_hbm.at[pl.ds(base + a * 128, 128)])

        nblk = ptrs[2 + h] + jnp.where(p > 0, 1, 0)
        cstg[pl.ds(0, 16)] = jnp.full((16,), 0, jnp.int32) + nblk
        pltpu.sync_copy(cstg, cnt_hbm.at[pl.ds((h * 32 + w) * 16, 16)])


@jax.jit
def _partition(packed):
    mesh = plsc.VectorSubcoreMesh(core_axis_name="c", subcore_axis_name="s")
    fn = pl.kernel(
        _part_body,
        mesh=mesh,
        out_type=(jax.ShapeDtypeStruct((PPACK_SIZE,), jnp.int32),
                  jax.ShapeDtypeStruct((1024,), jnp.int32)),
        scratch_types=[
            pltpu.VMEM((3, 128), jnp.int32),
            pltpu.VMEM((256,), jnp.int32),
            pltpu.VMEM((256,), jnp.int32),
            pltpu.VMEM((256,), jnp.int32),
            pltpu.VMEM((256,), jnp.int32),
            pltpu.VMEM((256,), jnp.int32),
            pltpu.VMEM((256,), jnp.int32),
            pltpu.VMEM((16,), jnp.int32),
            pltpu.SMEM((4,), jnp.int32),
        ],
        compiler_params=pltpu.CompilerParams(use_tc_tiling_on_sc=False,
                                             needs_layout_passes=False),
    )
    return fn(packed)


def _layer_body(egob_hbm, ppack_hbm, cnt_hbm, out_hbm,
                pbufA, pbufB, pbufC, rbA, rbB, rbC, rowsA, rowsB,
                dstlA, dstlB, dstlC, cb1, cb2,
                acc, semA, semB, semC):
    c = lax.axis_index("c")
    s = lax.axis_index("s")
    coff = c * HALF

    pbuf = (pbufA, pbufB, pbufC)
    rb = (rbA, rbB, rbC)
    rows = (rowsA, rowsB)
    dstl = (dstlA, dstlB, dstlC)
    sem = (semA, semB, semC)

    pltpu.sync_copy(cnt_hbm.at[pl.ds((c * 32 + s) * 16, 16)], cb1)
    pltpu.sync_copy(cnt_hbm.at[pl.ds((c * 32 + s + 16) * 16, 16)], cb2)
    n1 = cb1[pl.ds(0, 16)][0]
    n2 = cb2[pl.ds(0, 16)][0]
    nt = n1 + n2
    base1 = (c * 32 + s) * BUCKET_STRIDE
    base2 = (c * 32 + s + 16) * BUCKET_STRIDE

    def blockbase(j):
        return pl.multiple_of(
            jnp.where(j < n1, base1 + j * 384, base2 + (j - n1) * 384), 128)

    def gather_handle(B):
        return pltpu.make_async_copy(egob_hbm.at[pbuf[B].at[pl.ds(0, 128)]],
                                     rb[B], sem[B])

    def stage_load(jn, B):
        @pl.when(jnp.int32(jn) < nt)
        def _():
            base = blockbase(jn)
            pltpu.sync_copy(ppack_hbm.at[pl.ds(base, 384)], pbuf[B])
            for i in range(8):
                dstl[B][pl.ds(i * 16, 16)] = pbuf[B][pl.ds(128 + i * 16, 16)]
            gather_handle(B).start()

    MASKHI = jnp.int32(-65536)

    def stage_proc(jn, B, F):
        @pl.when(jnp.int32(jn) < nt)
        def _():
            gather_handle(B).wait()
            for i in range(8):
                vv = plsc.bitcast(pbuf[B][pl.ds(256 + i * 16, 16)],
                                  jnp.float32)
                for l in range(16):
                    r = i * 16 + l
                    v = vv[l]
                    p0 = rb[B][r, pl.ds(0, 16)]
                    p1 = rb[B][r, pl.ds(16, 16)]
                    rows[F][r, pl.ds(0, 16)] = (
                        plsc.bitcast(lax.shift_left(p0, 16), jnp.float32) * v)
                    rows[F][r, pl.ds(16, 16)] = (
                        plsc.bitcast(lax.shift_left(p1, 16), jnp.float32) * v)
                    rows[F][r, pl.ds(32, 16)] = (
                        plsc.bitcast(p0 & MASKHI, jnp.float32) * v)
                    rows[F][r, pl.ds(48, 16)] = (
                        plsc.bitcast(p1 & MASKHI, jnp.float32) * v)
            pltpu.sync_copy(rows[F], acc.at[dstl[B]], add=True)

    stage_load(0, 0)
    stage_load(1, 1)

    z = jnp.zeros((16,), jnp.float32)

    def zero_body(g, carry):
        for r in range(16):
            for q in range(D // 16):
                rowsA[g * 16 + r, pl.ds(q * 16, 16)] = z
        return carry

    lax.fori_loop(0, 8, zero_body, 0)
    for m in range(12):
        pltpu.sync_copy(rowsA, acc.at[pl.ds(s * STRIPE + m * 128, 128)])
    pltpu.sync_copy(rowsA.at[pl.ds(0, 32)],
                    acc.at[pl.ds(s * STRIPE + 1536, 32)])
    plsc.subcore_barrier()

    def ring_body(t, carry):
        stage_load(3 * t + 2, 2)
        stage_proc(3 * t, 0, 0)
        stage_load(3 * t + 3, 0)
        stage_proc(3 * t + 1, 1, 1)
        stage_load(3 * t + 4, 1)
        stage_proc(3 * t + 2, 2, 0)
        return carry

    lax.fori_loop(0, (nt + 2) // 3, ring_body, 0)
    plsc.subcore_barrier()

    st = s * STRIPE

    @pl.when(s < 15)
    def _():
        pltpu.sync_copy(acc.at[pl.ds(st, STRIPE)],
                        out_hbm.at[pl.ds(coff + st, STRIPE)])

    @pl.when(s == 15)
    def _():
        pltpu.sync_copy(acc.at[pl.ds(15 * STRIPE, LAST_STRIPE)],
                        out_hbm.at[pl.ds(coff + 15 * STRIPE, LAST_STRIPE)])


@jax.jit
def _propagate_layer(egob, ppack, counts):
    mesh = plsc.VectorSubcoreMesh(core_axis_name="c", subcore_axis_name="s")
    fn = pl.kernel(
        _layer_body,
        mesh=mesh,
        out_type=jax.ShapeDtypeStruct((N_NODES, D), jnp.float32),
        scratch_types=[
            pltpu.VMEM((384,), jnp.int32),
            pltpu.VMEM((384,), jnp.int32),
            pltpu.VMEM((384,), jnp.int32),
            pltpu.VMEM((128, 32), jnp.int32),
            pltpu.VMEM((128, 32), jnp.int32),
            pltpu.VMEM((128, 32), jnp.int32),
            pltpu.VMEM((128, D), jnp.float32),
            pltpu.VMEM((128, D), jnp.float32),
            pltpu.VMEM((128,), jnp.int32),
            pltpu.VMEM((128,), jnp.int32),
            pltpu.VMEM((128,), jnp.int32),
            pltpu.VMEM((16,), jnp.int32),
            pltpu.VMEM((16,), jnp.int32),
            pltpu.VMEM_SHARED((ACC_ROWS, D), jnp.float32),
            pltpu.SemaphoreType.DMA,
            pltpu.SemaphoreType.DMA,
            pltpu.SemaphoreType.DMA,
        ],
        compiler_params=pltpu.CompilerParams(use_tc_tiling_on_sc=False,
                                             needs_layout_passes=False),
    )
    return fn(egob, ppack, counts)


def _mean_body(a, b, c, d, o):
    o[...] = (a[...] + b[...] + c[...] + d[...]) * 0.25


@jax.jit
def _mean4(a, b, c, d):
    spec = pl.BlockSpec((400, D), lambda i: (i, 0))
    return pl.pallas_call(
        _mean_body,
        grid=(N_NODES // 400,),
        in_specs=[spec] * 4,
        out_specs=spec,
        out_shape=jax.ShapeDtypeStruct((N_NODES, D), jnp.float32),
    )(a, b, c, d)


def _pack_bf16_body(x, o):
    xb = x[...].astype(jnp.bfloat16)
    b16 = jax.lax.bitcast_convert_type(xb, jnp.uint16).astype(jnp.uint32)
    p = b16[:, :32] | (b16[:, 32:] << 16)
    o[...] = p.astype(jnp.int32)


@jax.jit
def _pack_bf16(x):
    return pl.pallas_call(
        _pack_bf16_body,
        grid=(N_NODES // 400,),
        in_specs=[pl.BlockSpec((400, D), lambda i: (i, 0))],
        out_specs=pl.BlockSpec((400, D // 2), lambda i: (i, 0)),
        out_shape=jax.ShapeDtypeStruct((N_NODES, D // 2), jnp.int32),
    )(x)


@jax.jit
def _pack_edges(adj_indices, adj_values):
    pad = E_PAD - N_EDGES
    src2d = jnp.concatenate(
        [adj_indices[0], jnp.zeros((pad,), jnp.int32)]).reshape(EROWS, 128)
    dst2d = jnp.concatenate(
        [adj_indices[1], jnp.zeros((pad,), jnp.int32)]).reshape(EROWS, 128)
    val2d = jnp.concatenate(
        [adj_values, jnp.zeros((pad,), jnp.float32)]
    ).reshape(EROWS, 128).view(jnp.int32)
    return jnp.stack([src2d, dst2d, val2d], axis=1).reshape(EROWS * 3, 128)


def kernel(user_emb, item_emb, adj_indices, adj_values):
    ego0 = jnp.concatenate([user_emb, item_emb], axis=0)
    packed = _pack_edges(adj_indices, adj_values)
    ppack, counts = _partition(packed)
    e1 = _propagate_layer(_pack_bf16(ego0), ppack, counts)
    e2 = _propagate_layer(_pack_bf16(e1), ppack, counts)
    e3 = _propagate_layer(_pack_bf16(e2), ppack, counts)
    final = _mean4(ego0, e1, e2, e3)
    return final[:N_USERS], final[N_USERS:]
